# baseline (speedup 1.0000x reference)
import jax
import jax.numpy as jnp
from jax.experimental import pallas as pl
from jax.experimental.pallas import tpu as pltpu


def _probe_kernel(x0_ref, x1_ref, x2_ref, o0_ref, o1_ref, o2_ref):
    for x_ref, o_ref in ((x0_ref, o0_ref), (x1_ref, o1_ref), (x2_ref, o2_ref)):
        x = x_ref[0]
        o_ref[0] = jnp.sum(x, axis=0, keepdims=True)


def kernel(fmap0, fmap1, fmap2, w0, w1, w2, b0, b1, b2):
    fmaps = [fmap0, fmap1, fmap2]
    B = fmap0.shape[0]
    R = w0.shape[0]
    Cs = [f.shape[1] for f in fmaps]
    Hs = [f.shape[2] for f in fmaps]
    Ws = [f.shape[3] for f in fmaps]
    NC = 2
    JB = B // NC

    def x_spec(c, h, w):
        return pl.BlockSpec((1, c, h, w), lambda i, j, JB=JB: (i * JB + j, 0, 0, 0))

    def o_spec(h, w):
        return pl.BlockSpec((1, 1, h, w), lambda i, j, JB=JB: (i * JB + j, 0, 0, 0))

    outs = pl.pallas_call(
        _probe_kernel,
        grid=(NC, JB),
        in_specs=[x_spec(Cs[0], Hs[0], Ws[0]),
                  x_spec(Cs[1], Hs[1], Ws[1]),
                  x_spec(Cs[2], Hs[2], Ws[2])],
        out_specs=[o_spec(Hs[0], Ws[0]), o_spec(Hs[1], Ws[1]), o_spec(Hs[2], Ws[2])],
        out_shape=[jax.ShapeDtypeStruct((B, 1, Hs[0], Ws[0]), jnp.float32),
                   jax.ShapeDtypeStruct((B, 1, Hs[1], Ws[1]), jnp.float32),
                   jax.ShapeDtypeStruct((B, 1, Hs[2], Ws[2]), jnp.float32)],
        compiler_params=pltpu.CompilerParams(
            dimension_semantics=("parallel", "arbitrary"),
            vmem_limit_bytes=64 * 1024 * 1024,
        ),
    )(fmaps[0], fmaps[1], fmaps[2])
    return outs


# speedup vs baseline: 2.0537x; 2.0537x over previous
import functools

import jax
import jax.numpy as jnp
import numpy as np
from jax import lax
from jax.experimental import pallas as pl
from jax.experimental.pallas import tpu as pltpu


def _pool_1d(n_in, n_out):
    p = np.zeros((n_in, n_out), np.float32)
    for i in range(n_out):
        s = (i * n_in) // n_out
        e = -(-((i + 1) * n_in) // n_out)
        p[s:e, i] = 1.0 / (e - s)
    return p


def _pool_matrix(in_hw, out_hw):
    hi, wi = in_hw
    ho, wo = out_hw
    ph = _pool_1d(hi, ho)
    pw = _pool_1d(wi, wo)
    return np.einsum("ih,jw->ijhw", ph, pw).reshape(hi * wi, ho * wo)


def _layer(x_ref, w_ref, brow_ref, bcol_ref, emb_ref, cert_ref, bi):
    x = x_ref[bi]
    hw = x.shape[1]
    inv_hw = jnp.float32(1.0 / hw)
    inv_hw_m1 = jnp.float32(1.0 / max(hw - 1, 1))

    mx = jnp.max(x, axis=-1, keepdims=True)
    s = jnp.sum(x, axis=-1, keepdims=True)
    pooled = (s * inv_hw + mx).astype(jnp.bfloat16)

    w = w_ref[...]
    emb = lax.dot_general(pooled, w, (((0,), (1,)), ((), ())),
                          preferred_element_type=jnp.float32)
    emb_ref[bi] = emb + brow_ref[...]

    xp = jnp.where(x == mx, x * jnp.float32(hw + 1), x).astype(jnp.bfloat16)
    cam = lax.dot_general(w, xp, (((1,), (0,)), ((), ())),
                          preferred_element_type=jnp.float32) + bcol_ref[...]

    m = jnp.sum(cam, axis=-1, keepdims=True) * inv_hw
    d = cam - m
    var = jnp.sum(d * d, axis=-1) * inv_hw_m1
    cert_ref[bi, 0] = jnp.sqrt(var)
    return cam


def _pooled_low(cam, p_ref):
    lp = lax.dot_general(cam.astype(jnp.bfloat16), p_ref[...],
                         (((1,), (0,)), ((), ())),
                         preferred_element_type=jnp.float32)
    inv = lax.rsqrt(jnp.maximum(
        jnp.sum(lp * lp, axis=-1, keepdims=True), 1e-24))
    return (lp * inv).astype(jnp.bfloat16)


def _link_g(low_n, cam_hi):
    inv_h = lax.rsqrt(jnp.maximum(
        jnp.sum(cam_hi * cam_hi, axis=-1, keepdims=True), 1e-24))
    hi_n = (cam_hi * inv_h).astype(jnp.bfloat16)
    return lax.dot_general(low_n, hi_n, (((1,), (1,)), ((), ())),
                           preferred_element_type=jnp.float32)


def _fused_kernel(x0_ref, x1_ref, x2_ref, w0_ref, w1_ref, w2_ref,
                  br0_ref, br1_ref, br2_ref, bc0_ref, bc1_ref, bc2_ref,
                  p0_ref, p1_ref,
                  emb0_ref, emb1_ref, emb2_ref,
                  cert0_ref, cert1_ref, cert2_ref,
                  l0_ref, l1_ref, *, inv_batch, bb):
    g0_sum = jnp.zeros((l0_ref.shape[1], l0_ref.shape[2]), jnp.float32)
    g1_sum = jnp.zeros_like(g0_sum)
    for bi in range(bb):
        cam0 = _layer(x0_ref, w0_ref, br0_ref, bc0_ref, emb0_ref, cert0_ref, bi)
        low0 = _pooled_low(cam0, p0_ref)

        cam1 = _layer(x1_ref, w1_ref, br1_ref, bc1_ref, emb1_ref, cert1_ref, bi)
        g0_sum = g0_sum + _link_g(low0, cam1)
        low1 = _pooled_low(cam1, p1_ref)

        cam2 = _layer(x2_ref, w2_ref, br2_ref, bc2_ref, emb2_ref, cert2_ref, bi)
        g1_sum = g1_sum + _link_g(low1, cam2)

    @pl.when(pl.program_id(1) == 0)
    def _():
        l0_ref[...] = jnp.zeros_like(l0_ref)
        l1_ref[...] = jnp.zeros_like(l1_ref)

    l0_ref[0] += g0_sum * jnp.float32(inv_batch)
    l1_ref[0] += g1_sum * jnp.float32(inv_batch)


def kernel(fmap0, fmap1, fmap2, w0, w1, w2, b0, b1, b2):
    fmaps = [fmap0, fmap1, fmap2]
    B = fmap0.shape[0]
    R = w0.shape[0]
    Cs = [f.shape[1] for f in fmaps]
    spatial = [(f.shape[2], f.shape[3]) for f in fmaps]
    HWs = [h * w for (h, w) in spatial]
    xs = [f.astype(jnp.float32).reshape(B, c, hw)
          for f, c, hw in zip(fmaps, Cs, HWs)]

    ws = [w.astype(jnp.bfloat16) for w in (w0, w1, w2)]
    brows = [b.reshape(1, R) for b in (b0, b1, b2)]
    bcols = [b.reshape(R, 1) for b in (b0, b1, b2)]
    p0 = jnp.asarray(_pool_matrix(spatial[0], spatial[1]), jnp.bfloat16)
    p1 = jnp.asarray(_pool_matrix(spatial[1], spatial[2]), jnp.bfloat16)

    NC = 2 if B % 2 == 0 else 1
    bb = 2 if (B // NC) % 2 == 0 else 1
    JB = B // (NC * bb)

    def x_spec(c, hw):
        return pl.BlockSpec((bb, c, hw), lambda i, j, JB=JB: (i * JB + j, 0, 0))

    def const_spec(shape):
        return pl.BlockSpec(shape, lambda i, j: (0,) * len(shape))

    def out_spec():
        return pl.BlockSpec((bb, 1, R), lambda i, j, JB=JB: (i * JB + j, 0, 0))

    br_shape = jax.ShapeDtypeStruct((B, 1, R), jnp.float32)
    outs = pl.pallas_call(
        functools.partial(_fused_kernel, inv_batch=1.0 / B, bb=bb),
        grid=(NC, JB),
        in_specs=[
            x_spec(Cs[0], HWs[0]), x_spec(Cs[1], HWs[1]), x_spec(Cs[2], HWs[2]),
            const_spec((R, Cs[0])), const_spec((R, Cs[1])), const_spec((R, Cs[2])),
            const_spec((1, R)), const_spec((1, R)), const_spec((1, R)),
            const_spec((R, 1)), const_spec((R, 1)), const_spec((R, 1)),
            const_spec((HWs[0], HWs[1])), const_spec((HWs[1], HWs[2])),
        ],
        out_specs=[
            out_spec(), out_spec(), out_spec(),
            out_spec(), out_spec(), out_spec(),
            pl.BlockSpec((1, R, R), lambda i, j: (i, 0, 0)),
            pl.BlockSpec((1, R, R), lambda i, j: (i, 0, 0)),
        ],
        out_shape=[
            br_shape, br_shape, br_shape,
            br_shape, br_shape, br_shape,
            jax.ShapeDtypeStruct((NC, R, R), jnp.float32),
            jax.ShapeDtypeStruct((NC, R, R), jnp.float32),
        ],
        compiler_params=pltpu.CompilerParams(
            dimension_semantics=("parallel", "arbitrary"),
            vmem_limit_bytes=64 * 1024 * 1024,
        ),
    )(xs[0], xs[1], xs[2], ws[0], ws[1], ws[2],
      brows[0], brows[1], brows[2], bcols[0], bcols[1], bcols[2], p0, p1)

    emb0, emb1, emb2, cert0, cert1, cert2, l0, l1 = outs
    embeddings = [e.reshape(B, R) for e in (emb0, emb1, emb2)]
    certainties = [c.reshape(B, R) for c in (cert0, cert1, cert2)]
    links = [jnp.sum(l0, axis=0), jnp.sum(l1, axis=0)]
    return embeddings, certainties, links
